# baseline (speedup 1.0000x reference)
import functools

import jax
import jax.numpy as jnp
from jax.experimental import pallas as pl
from jax.experimental.pallas import tpu as pltpu


def _conv_body(x_ref, w_ref, o_ref, *, R, W):
    x = x_ref[0, 0]
    acc = None
    for dy in range(3):
        for dx in range(3):
            lhs = x[dy:dy + R, dx:dx + W, :].reshape(R * W, x.shape[-1])
            p = jnp.dot(lhs, w_ref[dy * 3 + dx],
                        preferred_element_type=jnp.float32)
            acc = p if acc is None else acc + p
    o_ref[0, 0] = acc.reshape(R, W, acc.shape[-1])


def _conv3x3(x_nhwc, w, R=28):
    N, H, W, Ci = x_nhwc.shape
    Co = w.shape[0]
    S = H // R
    xp = jnp.pad(x_nhwc, ((0, 0), (1, 1), (1, 1), (0, 0)))
    xs = jnp.stack([xp[:, s * R:s * R + R + 2] for s in range(S)], axis=1)
    wt = jnp.transpose(w, (2, 3, 1, 0)).reshape(9, Ci, Co)
    out = pl.pallas_call(
        functools.partial(_conv_body, R=R, W=W),
        grid=(N, S),
        in_specs=[
            pl.BlockSpec((1, 1, R + 2, W + 2, Ci),
                         lambda n, s: (n, s, 0, 0, 0)),
            pl.BlockSpec((9, Ci, Co), lambda n, s: (0, 0, 0)),
        ],
        out_specs=pl.BlockSpec((1, 1, R, W, Co),
                               lambda n, s: (n, s, 0, 0, 0)),
        out_shape=jax.ShapeDtypeStruct((N, S, R, W, Co), jnp.float32),
        compiler_params=pltpu.CompilerParams(
            dimension_semantics=("parallel", "arbitrary"),
        ),
        name="conv3x3",
    )(xs, wt)
    return out.reshape(N, H, W, Co)


def kernel(x, w1, w2, w3):
    xt = jnp.transpose(x, (0, 2, 3, 1))
    y = _conv3x3(xt, w1)
    y = _conv3x3(y, w2)
    y = _conv3x3(y, w3)
    return jnp.transpose(y, (0, 3, 1, 2))
